# baseline (speedup 1.0000x reference)
import functools

import jax
import jax.numpy as jnp
from jax import lax
from jax.experimental import pallas as pl
from jax.experimental.pallas import tpu as pltpu
from jax.experimental.pallas import tpu_sc as plsc

BATCH = 16384
HIDDEN = 64
N_USERS = 1000000
N_ITEMS = 100000

NUM_CORES = 2
NUM_SUBCORES = 16
NUM_WORKERS = NUM_CORES * NUM_SUBCORES
B_PER_W = BATCH // NUM_WORKERS
CHUNK = 64
N_CHUNKS = B_PER_W // CHUNK
LANES = 16

_SC_MESH = plsc.VectorSubcoreMesh(core_axis_name="c", subcore_axis_name="s")


def _build_offsets(idx_v, j, offs, table_rows):
    for g in range(CHUNK // LANES):
        base = idx_v[0, pl.ds(j * CHUNK + g * LANES, LANES)]

        @pl.loop(0, HIDDEN)
        def _(d, _base=base, _g=g):
            offs[d, pl.ds(_g * LANES, LANES)] = _base + d * table_rows


@functools.partial(
    pl.kernel,
    mesh=_SC_MESH,
    out_type=[
        jax.ShapeDtypeStruct((HIDDEN, BATCH), jnp.float32),
        jax.ShapeDtypeStruct((HIDDEN, BATCH), jnp.float32),
    ],
    scratch_types=[
        pltpu.VMEM((1, B_PER_W), jnp.int32),
        pltpu.VMEM((1, B_PER_W), jnp.int32),
        pltpu.VMEM((HIDDEN, CHUNK), jnp.int32),
        pltpu.VMEM((HIDDEN, CHUNK), jnp.int32),
        pltpu.VMEM((HIDDEN, CHUNK), jnp.int32),
        pltpu.VMEM((HIDDEN, CHUNK), jnp.int32),
        pltpu.VMEM((HIDDEN, CHUNK), jnp.float32),
        pltpu.VMEM((HIDDEN, CHUNK), jnp.float32),
        pltpu.VMEM((HIDDEN, CHUNK), jnp.float32),
        pltpu.VMEM((HIDDEN, CHUNK), jnp.float32),
        pltpu.SemaphoreType.DMA,
        pltpu.SemaphoreType.DMA,
        pltpu.SemaphoreType.DMA,
        pltpu.SemaphoreType.DMA,
    ],
    compiler_params=pltpu.CompilerParams(use_tc_tiling_on_sc=False),
)
def _sc_gather(u_idx_hbm, i_idx_hbm, ut_hbm, it_hbm, uo_hbm, io_hbm,
               uidx_v, iidx_v, uoffs0, uoffs1, ioffs0, ioffs1,
               urows0, urows1, irows0, irows1,
               sem_u0, sem_u1, sem_i0, sem_i1):
    wid = lax.axis_index("s") * NUM_CORES + lax.axis_index("c")
    base = wid * B_PER_W
    pltpu.sync_copy(u_idx_hbm.at[pl.ds(wid, 1)], uidx_v)
    pltpu.sync_copy(i_idx_hbm.at[pl.ds(wid, 1)], iidx_v)

    uoffs = (uoffs0, uoffs1)
    ioffs = (ioffs0, ioffs1)
    ubufs = (urows0, urows1)
    ibufs = (irows0, irows1)
    usems = (sem_u0, sem_u1)
    isems = (sem_i0, sem_i1)
    def _drain(buf, sem):
        pltpu.make_async_copy(uo_hbm.at[:, pl.ds(0, CHUNK)], buf, sem).wait()

    for j in range(N_CHUNKS):
        b = j % 2
        _build_offsets(uidx_v, j, uoffs[b], N_USERS)
        _build_offsets(iidx_v, j, ioffs[b], N_ITEMS)

        @pl.loop(0, HIDDEN)
        def _(d, _b=b):
            pltpu.async_copy(ut_hbm.at[uoffs[_b].at[d]], ubufs[_b].at[d],
                             usems[_b])
            pltpu.async_copy(it_hbm.at[ioffs[_b].at[d]], ibufs[_b].at[d],
                             isems[_b])

        if j > 0:
            p = (j - 1) % 2
            _drain(ubufs[p], usems[p])
            _drain(ibufs[p], isems[p])
            col = pl.ds(base + (j - 1) * CHUNK, CHUNK)
            pltpu.sync_copy(ubufs[p], uo_hbm.at[:, col])
            pltpu.sync_copy(ibufs[p], io_hbm.at[:, col])
    lb = (N_CHUNKS - 1) % 2
    _drain(ubufs[lb], usems[lb])
    _drain(ibufs[lb], isems[lb])
    col = pl.ds(base + (N_CHUNKS - 1) * CHUNK, CHUNK)
    pltpu.sync_copy(ubufs[lb], uo_hbm.at[:, col])
    pltpu.sync_copy(ibufs[lb], io_hbm.at[:, col])


_MLP_BLOCK = 2048


def _mlp_body(ut, it, w1u, w1v, b1, w2, b2, w3, b3, o):
    h = (jnp.dot(w1u[...], ut[...], preferred_element_type=jnp.float32)
         + jnp.dot(w1v[...], it[...], preferred_element_type=jnp.float32)
         + b1[...])
    h = jnp.maximum(h, 0.0)
    h = jnp.dot(w2[...], h, preferred_element_type=jnp.float32) + b2[...]
    h = jnp.maximum(h, 0.0)
    z = jnp.dot(w3[...], h, preferred_element_type=jnp.float32) + b3[...]
    o[...] = jax.nn.sigmoid(z)


def _mlp(ut, it, w1u, w1v, b1, w2, b2, w3, b3):
    nb = BATCH // _MLP_BLOCK
    const = lambda *_: (0, 0)
    return pl.pallas_call(
        _mlp_body,
        grid=(nb,),
        in_specs=[
            pl.BlockSpec((HIDDEN, _MLP_BLOCK), lambda i: (0, i)),
            pl.BlockSpec((HIDDEN, _MLP_BLOCK), lambda i: (0, i)),
            pl.BlockSpec((HIDDEN, HIDDEN), const),
            pl.BlockSpec((HIDDEN, HIDDEN), const),
            pl.BlockSpec((HIDDEN, 1), const),
            pl.BlockSpec((HIDDEN // 2, HIDDEN), const),
            pl.BlockSpec((HIDDEN // 2, 1), const),
            pl.BlockSpec((1, HIDDEN // 2), const),
            pl.BlockSpec((1, 1), const),
        ],
        out_specs=pl.BlockSpec((1, _MLP_BLOCK), lambda i: (0, i)),
        out_shape=jax.ShapeDtypeStruct((1, BATCH), jnp.float32),
    )(ut, it, w1u, w1v, b1, w2, b2, w3, b3)


def kernel(user, item, user_table, item_table, W1, b1, W2, b2, W3, b3):
    u2d = user.astype(jnp.int32).reshape(NUM_WORKERS, B_PER_W)
    i2d = item.astype(jnp.int32).reshape(NUM_WORKERS, B_PER_W)
    ut1 = user_table.T.reshape(-1)
    it1 = item_table.T.reshape(-1)
    ut_t, it_t = _sc_gather(u2d, i2d, ut1, it1)
    w1t = W1.T
    w1u = w1t[:, :HIDDEN]
    w1v = w1t[:, HIDDEN:]
    z = _mlp(ut_t, it_t, w1u, w1v,
             b1.reshape(HIDDEN, 1), W2.T, b2.reshape(HIDDEN // 2, 1),
             W3.T, b3.reshape(1, 1))
    return z.reshape(BATCH, 1)


# speedup vs baseline: 5.6592x; 5.6592x over previous
import functools

import jax
import jax.numpy as jnp
from jax import lax
from jax.experimental import pallas as pl
from jax.experimental.pallas import tpu as pltpu
from jax.experimental.pallas import tpu_sc as plsc

BATCH = 16384
HIDDEN = 64

NUM_SC = 2
B_PER_C = BATCH // NUM_SC
IDX_CHUNK = 2048
N_ICHUNKS = B_PER_C // IDX_CHUNK

_SC_MESH = plsc.ScalarSubcoreMesh(axis_name="core", num_cores=NUM_SC)


@functools.partial(
    pl.kernel,
    mesh=_SC_MESH,
    out_type=[
        jax.ShapeDtypeStruct((BATCH, HIDDEN), jnp.float32),
        jax.ShapeDtypeStruct((BATCH, HIDDEN), jnp.float32),
    ],
    scratch_types=[
        pltpu.SMEM((IDX_CHUNK,), jnp.int32),
        pltpu.SMEM((IDX_CHUNK,), jnp.int32),
        pltpu.SemaphoreType.DMA,
        pltpu.SemaphoreType.DMA,
        pltpu.SemaphoreType.DMA,
    ],
)
def _sc_gather(u_idx_hbm, i_idx_hbm, ut_hbm, it_hbm, uo_hbm, io_hbm,
               us_s, is_s, sem_idx, sem_u, sem_i):
    cid = lax.axis_index("core")
    base = cid * B_PER_C

    for j in range(N_ICHUNKS):
        off = base + j * IDX_CHUNK
        pltpu.async_copy(u_idx_hbm.at[pl.ds(off, IDX_CHUNK)], us_s, sem_idx
                         ).wait()
        pltpu.async_copy(i_idx_hbm.at[pl.ds(off, IDX_CHUNK)], is_s, sem_idx
                         ).wait()

        @pl.loop(0, IDX_CHUNK)
        def _(r, _off=off):
            col = _off + r
            u = us_s[r]
            i = is_s[r]
            pltpu.async_copy(ut_hbm.at[pl.ds(u, 1)],
                             uo_hbm.at[pl.ds(col, 1)], sem_u)
            pltpu.async_copy(it_hbm.at[pl.ds(i, 1)],
                             io_hbm.at[pl.ds(col, 1)], sem_i)

    pltpu.make_async_copy(ut_hbm.at[pl.ds(0, B_PER_C)],
                          uo_hbm.at[pl.ds(base, B_PER_C)], sem_u).wait()
    pltpu.make_async_copy(it_hbm.at[pl.ds(0, B_PER_C)],
                          io_hbm.at[pl.ds(base, B_PER_C)], sem_i).wait()


_MLP_BLOCK = 2048


def _mlp_body(ue, ie, w1u, w1v, b1, w2, b2, w3, b3, o):
    h = (jnp.dot(ue[...], w1u[...], preferred_element_type=jnp.float32)
         + jnp.dot(ie[...], w1v[...], preferred_element_type=jnp.float32)
         + b1[...])
    h = jnp.maximum(h, 0.0)
    h = jnp.dot(h, w2[...], preferred_element_type=jnp.float32) + b2[...]
    h = jnp.maximum(h, 0.0)
    z = jnp.dot(h, w3[...], preferred_element_type=jnp.float32) + b3[...]
    o[...] = jax.nn.sigmoid(z)


def _mlp(ue, ie, w1u, w1v, b1, w2, b2, w3, b3):
    nb = BATCH // _MLP_BLOCK
    const = lambda *_: (0, 0)
    return pl.pallas_call(
        _mlp_body,
        grid=(nb,),
        in_specs=[
            pl.BlockSpec((_MLP_BLOCK, HIDDEN), lambda i: (i, 0)),
            pl.BlockSpec((_MLP_BLOCK, HIDDEN), lambda i: (i, 0)),
            pl.BlockSpec((HIDDEN, HIDDEN), const),
            pl.BlockSpec((HIDDEN, HIDDEN), const),
            pl.BlockSpec((1, HIDDEN), const),
            pl.BlockSpec((HIDDEN, HIDDEN // 2), const),
            pl.BlockSpec((1, HIDDEN // 2), const),
            pl.BlockSpec((HIDDEN // 2, 1), const),
            pl.BlockSpec((1, 1), const),
        ],
        out_specs=pl.BlockSpec((_MLP_BLOCK, 1), lambda i: (i, 0)),
        out_shape=jax.ShapeDtypeStruct((BATCH, 1), jnp.float32),
    )(ue, ie, w1u, w1v, b1, w2, b2, w3, b3)


def kernel(user, item, user_table, item_table, W1, b1, W2, b2, W3, b3):
    user = user.astype(jnp.int32)
    item = item.astype(jnp.int32)
    ue, ie = _sc_gather(user, item, user_table, item_table)
    w1u = W1[:HIDDEN]
    w1v = W1[HIDDEN:]
    return _mlp(ue, ie, w1u, w1v,
                b1.reshape(1, HIDDEN), W2, b2.reshape(1, HIDDEN // 2),
                W3, b3.reshape(1, 1))


# speedup vs baseline: 8.7072x; 1.5386x over previous
import functools

import jax
import jax.numpy as jnp
from jax import lax
from jax.experimental import pallas as pl
from jax.experimental.pallas import tpu as pltpu
from jax.experimental.pallas import tpu_sc as plsc

BATCH = 16384
HIDDEN = 64
WIDE = 2 * HIDDEN

NUM_CORES = 2
NUM_SUBCORES = 16
NUM_WORKERS = NUM_CORES * NUM_SUBCORES
B_PER_W = BATCH // NUM_WORKERS
CHUNK = 64
N_CHUNKS = B_PER_W // CHUNK

_SC_MESH = plsc.VectorSubcoreMesh(core_axis_name="c", subcore_axis_name="s")


_PRJ_BLOCK = 2048


def _project_body(tt, w, o):
    o[...] = jax.lax.dot_general(tt[...], w[...], (((0,), (0,)), ((), ())),
                                 preferred_element_type=jnp.float32)


def _project(tt, w, n_rows):
    nb = pl.cdiv(n_rows, _PRJ_BLOCK)
    return pl.pallas_call(
        _project_body,
        grid=(nb,),
        in_specs=[
            pl.BlockSpec((HIDDEN, _PRJ_BLOCK), lambda i: (0, i)),
            pl.BlockSpec((HIDDEN, WIDE), lambda i: (0, 0)),
        ],
        out_specs=pl.BlockSpec((_PRJ_BLOCK, WIDE), lambda i: (i, 0)),
        out_shape=jax.ShapeDtypeStruct((n_rows, WIDE), jnp.float32),
    )(tt, w)


@functools.partial(
    pl.kernel,
    mesh=_SC_MESH,
    out_type=[
        jax.ShapeDtypeStruct((BATCH, WIDE), jnp.float32),
        jax.ShapeDtypeStruct((BATCH, WIDE), jnp.float32),
    ],
    scratch_types=[
        pltpu.VMEM((N_CHUNKS, CHUNK), jnp.int32),
        pltpu.VMEM((N_CHUNKS, CHUNK), jnp.int32),
        pltpu.VMEM((CHUNK, WIDE), jnp.float32),
        pltpu.VMEM((CHUNK, WIDE), jnp.float32),
        pltpu.VMEM((CHUNK, WIDE), jnp.float32),
        pltpu.VMEM((CHUNK, WIDE), jnp.float32),
        pltpu.SemaphoreType.DMA,
        pltpu.SemaphoreType.DMA,
        pltpu.SemaphoreType.DMA,
        pltpu.SemaphoreType.DMA,
    ],
)
def _sc_gather(u_idx_hbm, i_idx_hbm, pu_hbm, pi_hbm, uo_hbm, io_hbm,
               uidx_v, iidx_v, urows0, urows1, irows0, irows1,
               sem_u0, sem_u1, sem_i0, sem_i1):
    wid = lax.axis_index("s") * NUM_CORES + lax.axis_index("c")
    base = wid * B_PER_W
    pltpu.sync_copy(u_idx_hbm.at[pl.ds(wid * N_CHUNKS, N_CHUNKS)], uidx_v)
    pltpu.sync_copy(i_idx_hbm.at[pl.ds(wid * N_CHUNKS, N_CHUNKS)], iidx_v)

    ubufs = (urows0, urows1)
    ibufs = (irows0, irows1)
    usems = (sem_u0, sem_u1)
    isems = (sem_i0, sem_i1)
    gathers = [None, None]
    for j in range(N_CHUNKS):
        b = j % 2
        gathers[b] = (
            pltpu.async_copy(pu_hbm.at[uidx_v.at[j]], ubufs[b], usems[b]),
            pltpu.async_copy(pi_hbm.at[iidx_v.at[j]], ibufs[b], isems[b]),
        )
        if j > 0:
            p = (j - 1) % 2
            gu, gi = gathers[p]
            gu.wait()
            gi.wait()
            off = base + (j - 1) * CHUNK
            pltpu.sync_copy(ubufs[p], uo_hbm.at[pl.ds(off, CHUNK)])
            pltpu.sync_copy(ibufs[p], io_hbm.at[pl.ds(off, CHUNK)])
    lb = (N_CHUNKS - 1) % 2
    gu, gi = gathers[lb]
    gu.wait()
    gi.wait()
    off = base + (N_CHUNKS - 1) * CHUNK
    pltpu.sync_copy(ubufs[lb], uo_hbm.at[pl.ds(off, CHUNK)])
    pltpu.sync_copy(ibufs[lb], io_hbm.at[pl.ds(off, CHUNK)])


_MLP_BLOCK = 2048


def _tail_body(pu, pi, b1, w2, b2, w3, b3, o):
    h = jnp.maximum(pu[...][:, :HIDDEN] + pi[...][:, HIDDEN:] + b1[...], 0.0)
    h = jnp.dot(h, w2[...], preferred_element_type=jnp.float32) + b2[...]
    h = jnp.maximum(h, 0.0)
    z = jnp.dot(h, w3[...], preferred_element_type=jnp.float32) + b3[...]
    o[...] = jax.nn.sigmoid(z)


def _tail(pu, pi, b1, w2, b2, w3, b3):
    nb = BATCH // _MLP_BLOCK
    const = lambda *_: (0, 0)
    return pl.pallas_call(
        _tail_body,
        grid=(nb,),
        in_specs=[
            pl.BlockSpec((_MLP_BLOCK, WIDE), lambda i: (i, 0)),
            pl.BlockSpec((_MLP_BLOCK, WIDE), lambda i: (i, 0)),
            pl.BlockSpec((1, HIDDEN), const),
            pl.BlockSpec((HIDDEN, HIDDEN // 2), const),
            pl.BlockSpec((1, HIDDEN // 2), const),
            pl.BlockSpec((HIDDEN // 2, 1), const),
            pl.BlockSpec((1, 1), const),
        ],
        out_specs=pl.BlockSpec((_MLP_BLOCK, 1), lambda i: (i, 0)),
        out_shape=jax.ShapeDtypeStruct((BATCH, 1), jnp.float32),
    )(pu, pi, b1, w2, b2, w3, b3)


def kernel(user, item, user_table, item_table, W1, b1, W2, b2, W3, b3):
    user = user.astype(jnp.int32)
    item = item.astype(jnp.int32)
    w1u_pad = jnp.pad(W1[:HIDDEN], ((0, 0), (0, HIDDEN)))
    w1v_pad = jnp.pad(W1[HIDDEN:], ((0, 0), (HIDDEN, 0)))
    pu_w = _project(user_table.T, w1u_pad, user_table.shape[0])
    pi_w = _project(item_table.T, w1v_pad, item_table.shape[0])
    u2d = user.reshape(BATCH // CHUNK, CHUNK)
    i2d = item.reshape(BATCH // CHUNK, CHUNK)
    gu, gi = _sc_gather(u2d, i2d, pu_w, pi_w)
    return _tail(gu, gi,
                 b1.reshape(1, HIDDEN), W2, b2.reshape(1, HIDDEN // 2),
                 W3, b3.reshape(1, 1))


# speedup vs baseline: 15.2995x; 1.7571x over previous
import functools

import jax
import jax.numpy as jnp
from jax import lax
from jax.experimental import pallas as pl
from jax.experimental.pallas import tpu as pltpu
from jax.experimental.pallas import tpu_sc as plsc

BATCH = 16384
HIDDEN = 64
WIDE = 2 * HIDDEN

NUM_CORES = 2
NUM_SUBCORES = 16
NUM_WORKERS = NUM_CORES * NUM_SUBCORES
B_PER_W = BATCH // NUM_WORKERS
CHUNK = 64
N_CHUNKS = B_PER_W // CHUNK

_SC_MESH = plsc.VectorSubcoreMesh(core_axis_name="c", subcore_axis_name="s")


_PRJ_BLOCK = 8192


def _project_body(tt, w, o):
    o[...] = jax.lax.dot_general(tt[...].astype(jnp.bfloat16),
                                 w[...].astype(jnp.bfloat16),
                                 (((0,), (0,)), ((), ())),
                                 preferred_element_type=jnp.float32)


def _project(tt, w, n_rows):
    nb = pl.cdiv(n_rows, _PRJ_BLOCK)
    return pl.pallas_call(
        _project_body,
        grid=(nb,),
        in_specs=[
            pl.BlockSpec((HIDDEN, _PRJ_BLOCK), lambda i: (0, i)),
            pl.BlockSpec((HIDDEN, WIDE), lambda i: (0, 0)),
        ],
        out_specs=pl.BlockSpec((_PRJ_BLOCK, WIDE), lambda i: (i, 0)),
        out_shape=jax.ShapeDtypeStruct((n_rows, WIDE), jnp.float32),
    )(tt, w)


@functools.partial(
    pl.kernel,
    mesh=_SC_MESH,
    out_type=[
        jax.ShapeDtypeStruct((BATCH, WIDE), jnp.float32),
        jax.ShapeDtypeStruct((BATCH, WIDE), jnp.float32),
    ],
    scratch_types=[
        pltpu.VMEM((N_CHUNKS, CHUNK), jnp.int32),
        pltpu.VMEM((N_CHUNKS, CHUNK), jnp.int32),
        pltpu.VMEM((CHUNK, WIDE), jnp.float32),
        pltpu.VMEM((CHUNK, WIDE), jnp.float32),
        pltpu.VMEM((CHUNK, WIDE), jnp.float32),
        pltpu.VMEM((CHUNK, WIDE), jnp.float32),
        pltpu.SemaphoreType.DMA,
        pltpu.SemaphoreType.DMA,
        pltpu.SemaphoreType.DMA,
        pltpu.SemaphoreType.DMA,
    ],
)
def _sc_gather(u_idx_hbm, i_idx_hbm, pu_hbm, pi_hbm, uo_hbm, io_hbm,
               uidx_v, iidx_v, urows0, urows1, irows0, irows1,
               sem_u0, sem_u1, sem_i0, sem_i1):
    wid = lax.axis_index("s") * NUM_CORES + lax.axis_index("c")
    base = wid * B_PER_W
    pltpu.sync_copy(u_idx_hbm.at[pl.ds(wid * N_CHUNKS, N_CHUNKS)], uidx_v)
    pltpu.sync_copy(i_idx_hbm.at[pl.ds(wid * N_CHUNKS, N_CHUNKS)], iidx_v)

    ubufs = (urows0, urows1)
    ibufs = (irows0, irows1)
    usems = (sem_u0, sem_u1)
    isems = (sem_i0, sem_i1)
    gathers = [None, None]
    for j in range(N_CHUNKS):
        b = j % 2
        gathers[b] = (
            pltpu.async_copy(pu_hbm.at[uidx_v.at[j]], ubufs[b], usems[b]),
            pltpu.async_copy(pi_hbm.at[iidx_v.at[j]], ibufs[b], isems[b]),
        )
        if j > 0:
            p = (j - 1) % 2
            gu, gi = gathers[p]
            gu.wait()
            gi.wait()
            off = base + (j - 1) * CHUNK
            pltpu.sync_copy(ubufs[p], uo_hbm.at[pl.ds(off, CHUNK)])
            pltpu.sync_copy(ibufs[p], io_hbm.at[pl.ds(off, CHUNK)])
    lb = (N_CHUNKS - 1) % 2
    gu, gi = gathers[lb]
    gu.wait()
    gi.wait()
    off = base + (N_CHUNKS - 1) * CHUNK
    pltpu.sync_copy(ubufs[lb], uo_hbm.at[pl.ds(off, CHUNK)])
    pltpu.sync_copy(ibufs[lb], io_hbm.at[pl.ds(off, CHUNK)])


_MLP_BLOCK = 2048


def _tail_body(pu, pi, b1, w2, b2, w3, b3, o):
    h = jnp.maximum(pu[...][:, :HIDDEN] + pi[...][:, HIDDEN:] + b1[...], 0.0)
    h = jnp.dot(h, w2[...], preferred_element_type=jnp.float32) + b2[...]
    h = jnp.maximum(h, 0.0)
    z = jnp.dot(h, w3[...], preferred_element_type=jnp.float32) + b3[...]
    o[...] = jax.nn.sigmoid(z)


def _tail(pu, pi, b1, w2, b2, w3, b3):
    nb = BATCH // _MLP_BLOCK
    const = lambda *_: (0, 0)
    return pl.pallas_call(
        _tail_body,
        grid=(nb,),
        in_specs=[
            pl.BlockSpec((_MLP_BLOCK, WIDE), lambda i: (i, 0)),
            pl.BlockSpec((_MLP_BLOCK, WIDE), lambda i: (i, 0)),
            pl.BlockSpec((1, HIDDEN), const),
            pl.BlockSpec((HIDDEN, HIDDEN // 2), const),
            pl.BlockSpec((1, HIDDEN // 2), const),
            pl.BlockSpec((HIDDEN // 2, 1), const),
            pl.BlockSpec((1, 1), const),
        ],
        out_specs=pl.BlockSpec((_MLP_BLOCK, 1), lambda i: (i, 0)),
        out_shape=jax.ShapeDtypeStruct((BATCH, 1), jnp.float32),
    )(pu, pi, b1, w2, b2, w3, b3)


def kernel(user, item, user_table, item_table, W1, b1, W2, b2, W3, b3):
    user = user.astype(jnp.int32)
    item = item.astype(jnp.int32)
    w1u_pad = jnp.pad(W1[:HIDDEN], ((0, 0), (0, HIDDEN)))
    w1v_pad = jnp.pad(W1[HIDDEN:], ((0, 0), (HIDDEN, 0)))
    pu_w = _project(user_table.T, w1u_pad, user_table.shape[0])
    pi_w = _project(item_table.T, w1v_pad, item_table.shape[0])
    u2d = user.reshape(BATCH // CHUNK, CHUNK)
    i2d = item.reshape(BATCH // CHUNK, CHUNK)
    gu, gi = _sc_gather(u2d, i2d, pu_w, pi_w)
    return _tail(gu, gi,
                 b1.reshape(1, HIDDEN), W2, b2.reshape(1, HIDDEN // 2),
                 W3, b3.reshape(1, 1))


# speedup vs baseline: 17.7141x; 1.1578x over previous
import functools

import jax
import jax.numpy as jnp
from jax import lax
from jax.experimental import pallas as pl
from jax.experimental.pallas import tpu as pltpu
from jax.experimental.pallas import tpu_sc as plsc

BATCH = 16384
HIDDEN = 64
WIDE = 2 * HIDDEN

NUM_CORES = 2
NUM_SUBCORES = 16
NUM_WORKERS = NUM_CORES * NUM_SUBCORES
B_PER_W = BATCH // NUM_WORKERS
CHUNK = 64
N_CHUNKS = B_PER_W // CHUNK

_SC_MESH = plsc.VectorSubcoreMesh(core_axis_name="c", subcore_axis_name="s")


_PRJ_BLOCK = 8192


def _project_body(tt, w, o):
    p = jax.lax.dot_general(tt[...].astype(jnp.bfloat16),
                            w[...].astype(jnp.bfloat16),
                            (((0,), (0,)), ((), ())),
                            preferred_element_type=jnp.float32)
    o[...] = pltpu.bitcast(p.astype(jnp.bfloat16), jnp.float32)


def _project(tt, w, n_rows):
    nb = pl.cdiv(n_rows, _PRJ_BLOCK)
    return pl.pallas_call(
        _project_body,
        grid=(nb,),
        in_specs=[
            pl.BlockSpec((HIDDEN, _PRJ_BLOCK), lambda i: (0, i)),
            pl.BlockSpec((HIDDEN, WIDE), lambda i: (0, 0)),
        ],
        out_specs=pl.BlockSpec((_PRJ_BLOCK // 2, WIDE), lambda i: (i, 0)),
        out_shape=jax.ShapeDtypeStruct((n_rows // 2, WIDE), jnp.float32),
    )(tt, w)


@functools.partial(
    pl.kernel,
    mesh=_SC_MESH,
    out_type=[
        jax.ShapeDtypeStruct((BATCH, WIDE), jnp.float32),
        jax.ShapeDtypeStruct((BATCH, WIDE), jnp.float32),
    ],
    scratch_types=[
        pltpu.VMEM((N_CHUNKS, CHUNK), jnp.int32),
        pltpu.VMEM((N_CHUNKS, CHUNK), jnp.int32),
        pltpu.VMEM((CHUNK, WIDE), jnp.float32),
        pltpu.VMEM((CHUNK, WIDE), jnp.float32),
        pltpu.VMEM((CHUNK, WIDE), jnp.float32),
        pltpu.VMEM((CHUNK, WIDE), jnp.float32),
        pltpu.SemaphoreType.DMA,
        pltpu.SemaphoreType.DMA,
        pltpu.SemaphoreType.DMA,
        pltpu.SemaphoreType.DMA,
    ],
)
def _sc_gather(u_idx_hbm, i_idx_hbm, pu_hbm, pi_hbm, uo_hbm, io_hbm,
               uidx_v, iidx_v, urows0, urows1, irows0, irows1,
               sem_u0, sem_u1, sem_i0, sem_i1):
    wid = lax.axis_index("s") * NUM_CORES + lax.axis_index("c")
    base = wid * B_PER_W
    pltpu.sync_copy(u_idx_hbm.at[pl.ds(wid * N_CHUNKS, N_CHUNKS)], uidx_v)
    pltpu.sync_copy(i_idx_hbm.at[pl.ds(wid * N_CHUNKS, N_CHUNKS)], iidx_v)

    ubufs = (urows0, urows1)
    ibufs = (irows0, irows1)
    usems = (sem_u0, sem_u1)
    isems = (sem_i0, sem_i1)
    gathers = [None, None]
    for j in range(N_CHUNKS):
        b = j % 2
        gathers[b] = (
            pltpu.async_copy(pu_hbm.at[uidx_v.at[j]], ubufs[b], usems[b]),
            pltpu.async_copy(pi_hbm.at[iidx_v.at[j]], ibufs[b], isems[b]),
        )
        if j > 0:
            p = (j - 1) % 2
            gu, gi = gathers[p]
            gu.wait()
            gi.wait()
            off = base + (j - 1) * CHUNK
            pltpu.sync_copy(ubufs[p], uo_hbm.at[pl.ds(off, CHUNK)])
            pltpu.sync_copy(ibufs[p], io_hbm.at[pl.ds(off, CHUNK)])
    lb = (N_CHUNKS - 1) % 2
    gu, gi = gathers[lb]
    gu.wait()
    gi.wait()
    off = base + (N_CHUNKS - 1) * CHUNK
    pltpu.sync_copy(ubufs[lb], uo_hbm.at[pl.ds(off, CHUNK)])
    pltpu.sync_copy(ibufs[lb], io_hbm.at[pl.ds(off, CHUNK)])


_MLP_BLOCK = 2048


def _unpack_select(packed, parity):
    g = pltpu.bitcast(packed, jnp.uint32)
    lo = pltpu.bitcast(g << jnp.uint32(16), jnp.float32)
    hi = pltpu.bitcast(g & jnp.uint32(0xFFFF0000), jnp.float32)
    return jnp.where(parity > 0, hi, lo)


def _tail_body(pu, pi, su, si, b1, w2, b2, w3, b3, o):
    eu = _unpack_select(pu[...][:, :HIDDEN], su[...])
    ei = _unpack_select(pi[...][:, HIDDEN:], si[...])
    h = jnp.maximum(eu + ei + b1[...], 0.0)
    h = jnp.dot(h, w2[...], preferred_element_type=jnp.float32) + b2[...]
    h = jnp.maximum(h, 0.0)
    z = jnp.dot(h, w3[...], preferred_element_type=jnp.float32) + b3[...]
    o[...] = jax.nn.sigmoid(z)


def _tail(pu, pi, su, si, b1, w2, b2, w3, b3):
    nb = BATCH // _MLP_BLOCK
    const = lambda *_: (0, 0)
    return pl.pallas_call(
        _tail_body,
        grid=(nb,),
        in_specs=[
            pl.BlockSpec((_MLP_BLOCK, WIDE), lambda i: (i, 0)),
            pl.BlockSpec((_MLP_BLOCK, WIDE), lambda i: (i, 0)),
            pl.BlockSpec((_MLP_BLOCK, 1), lambda i: (i, 0)),
            pl.BlockSpec((_MLP_BLOCK, 1), lambda i: (i, 0)),
            pl.BlockSpec((1, HIDDEN), const),
            pl.BlockSpec((HIDDEN, HIDDEN // 2), const),
            pl.BlockSpec((1, HIDDEN // 2), const),
            pl.BlockSpec((HIDDEN // 2, 1), const),
            pl.BlockSpec((1, 1), const),
        ],
        out_specs=pl.BlockSpec((_MLP_BLOCK, 1), lambda i: (i, 0)),
        out_shape=jax.ShapeDtypeStruct((BATCH, 1), jnp.float32),
    )(pu, pi, su, si, b1, w2, b2, w3, b3)


def kernel(user, item, user_table, item_table, W1, b1, W2, b2, W3, b3):
    user = user.astype(jnp.int32)
    item = item.astype(jnp.int32)
    w1u_pad = jnp.pad(W1[:HIDDEN], ((0, 0), (0, HIDDEN)))
    w1v_pad = jnp.pad(W1[HIDDEN:], ((0, 0), (HIDDEN, 0)))
    pu_w = _project(user_table.T, w1u_pad, user_table.shape[0])
    pi_w = _project(item_table.T, w1v_pad, item_table.shape[0])
    u2d = (user // 2).reshape(BATCH // CHUNK, CHUNK)
    i2d = (item // 2).reshape(BATCH // CHUNK, CHUNK)
    gu, gi = _sc_gather(u2d, i2d, pu_w, pi_w)
    su = (user % 2).astype(jnp.float32).reshape(BATCH, 1)
    si = (item % 2).astype(jnp.float32).reshape(BATCH, 1)
    return _tail(gu, gi, su, si,
                 b1.reshape(1, HIDDEN), W2, b2.reshape(1, HIDDEN // 2),
                 W3, b3.reshape(1, 1))
